# Initial kernel scaffold; baseline (speedup 1.0000x reference)
import jax
import jax.numpy as jnp
from jax.experimental import pallas as pl


def kernel(unknown, known, unknow_feats, known_feats, W1, b1, gamma1, beta1, W2, b2, gamma2, beta2):
    raise NotImplementedError("write your pallas kernel here")


# speedup vs baseline: 17.7394x; 17.7394x over previous
import jax
import jax.numpy as jnp
from jax.experimental import pallas as pl
from jax.experimental.pallas import tpu as pltpu

_B, _N, _M = 16, 4096, 1024
_C1, _C2 = 256, 512
_TN = 512
_NT = _N // _TN
_EPS_BN = 1e-5
_CNT = float(_B * _N)


def _k1(u_ref, kt_ref, kf_ref, uf_ref, w1a_ref, w1b_ref, b1_ref,
        y1_ref, st_ref):
    b = pl.program_id(0)
    nt = pl.program_id(1)

    u = u_ref[0]
    kt = kt_ref[0]
    d2 = jnp.zeros((_TN, _M), jnp.float32)
    for c in range(3):
        diff = u[:, c:c + 1] - kt[c:c + 1, :]
        d2 = d2 + diff * diff

    iota = jax.lax.broadcasted_iota(jnp.int32, (_TN, _M), 1)
    cur = d2
    recips = []
    idxs = []
    for _ in range(3):
        m = jnp.min(cur, axis=1, keepdims=True)
        cand = jnp.where(cur <= m, iota, _M)
        idxk = jnp.min(cand, axis=1, keepdims=True)
        cur = jnp.where(iota == idxk, jnp.float32(jnp.inf), cur)
        dist = jnp.sqrt(jnp.maximum(m, 0.0))
        recips.append(1.0 / (dist + 1e-8))
        idxs.append(idxk)
    rsum = recips[0] + recips[1] + recips[2]
    zero = jnp.zeros((), jnp.float32)
    s = jnp.where(iota == idxs[0], recips[0] / rsum, zero)
    s = s + jnp.where(iota == idxs[1], recips[1] / rsum, zero)
    s = s + jnp.where(iota == idxs[2], recips[2] / rsum, zero)
    sb = s.astype(jnp.bfloat16)

    kf = kf_ref[0]
    interp = jax.lax.dot_general(sb, kf, (((1,), (1,)), ((), ())),
                                 preferred_element_type=jnp.float32)
    y1 = jax.lax.dot_general(interp.astype(jnp.bfloat16), w1a_ref[...],
                             (((1,), (1,)), ((), ())),
                             preferred_element_type=jnp.float32)
    uf = uf_ref[0]
    y1 = y1 + jax.lax.dot_general(uf, w1b_ref[...],
                                  (((0,), (1,)), ((), ())),
                                  preferred_element_type=jnp.float32)
    y1 = y1 + b1_ref[...]

    @pl.when(jnp.logical_and(b == 0, nt == 0))
    def _():
        st_ref[...] = jnp.zeros_like(st_ref)

    st_ref[0:1, :] += jnp.sum(y1, axis=0, keepdims=True)
    st_ref[1:2, :] += jnp.sum(y1 * y1, axis=0, keepdims=True)
    y1_ref[0] = y1.astype(jnp.bfloat16)


def _k2(y1_ref, st1_ref, g1_ref, be1_ref, w2_ref, b2_ref,
        y2_ref, st_ref):
    b = pl.program_id(0)
    nt = pl.program_id(1)

    mean = st1_ref[0:1, :] * (1.0 / _CNT)
    var = st1_ref[1:2, :] * (1.0 / _CNT) - mean * mean
    a1 = g1_ref[...] * jax.lax.rsqrt(var + _EPS_BN)
    c1 = be1_ref[...] - mean * a1

    y1 = y1_ref[0].astype(jnp.float32)
    h1 = jnp.maximum(a1 * y1 + c1, 0.0).astype(jnp.bfloat16)
    y2 = jax.lax.dot_general(w2_ref[...], h1, (((1,), (1,)), ((), ())),
                             preferred_element_type=jnp.float32)
    y2 = y2 + b2_ref[...]

    @pl.when(jnp.logical_and(b == 0, nt == 0))
    def _():
        st_ref[...] = jnp.zeros_like(st_ref)

    st_ref[:, 0:1] += jnp.sum(y2, axis=1, keepdims=True)
    st_ref[:, 1:2] += jnp.sum(y2 * y2, axis=1, keepdims=True)
    y2_ref[0] = y2.astype(jnp.bfloat16)


def _k3(y2_ref, st2_ref, g2_ref, be2_ref, out_ref):
    mean = st2_ref[:, 0:1] * (1.0 / _CNT)
    var = st2_ref[:, 1:2] * (1.0 / _CNT) - mean * mean
    a2 = g2_ref[...] * jax.lax.rsqrt(var + _EPS_BN)
    c2 = be2_ref[...] - mean * a2
    y2 = y2_ref[0].astype(jnp.float32)
    out_ref[0] = jnp.maximum(a2 * y2 + c2, 0.0)


def kernel(unknown, known, unknow_feats, known_feats,
           W1, b1, gamma1, beta1, W2, b2, gamma2, beta2):
    known_t = jnp.swapaxes(known, 1, 2)
    kf_b = known_feats.astype(jnp.bfloat16)
    uf_b = unknow_feats.astype(jnp.bfloat16)
    w1a = W1[:, :_C2].astype(jnp.bfloat16)
    w1b = W1[:, _C2:].astype(jnp.bfloat16)
    w2 = W2.astype(jnp.bfloat16)
    b1r = b1.reshape(1, _C2)
    g1r = gamma1.reshape(1, _C2)
    be1r = beta1.reshape(1, _C2)
    b2r = b2.reshape(_C2, 1)
    g2r = gamma2.reshape(_C2, 1)
    be2r = beta2.reshape(_C2, 1)

    const2 = lambda bs: pl.BlockSpec(bs, lambda b, n: (0, 0))

    y1t, st1 = pl.pallas_call(
        _k1,
        grid=(_B, _NT),
        in_specs=[
            pl.BlockSpec((1, _TN, 3), lambda b, n: (b, n, 0)),
            pl.BlockSpec((1, 3, _M), lambda b, n: (b, 0, 0)),
            pl.BlockSpec((1, _C2, _M), lambda b, n: (b, 0, 0)),
            pl.BlockSpec((1, _C1, _TN), lambda b, n: (b, 0, n)),
            const2((_C2, _C2)),
            const2((_C2, _C1)),
            const2((1, _C2)),
        ],
        out_specs=[
            pl.BlockSpec((1, _TN, _C2), lambda b, n: (b, n, 0)),
            const2((2, _C2)),
        ],
        out_shape=[
            jax.ShapeDtypeStruct((_B, _N, _C2), jnp.bfloat16),
            jax.ShapeDtypeStruct((2, _C2), jnp.float32),
        ],
    )(unknown, known_t, kf_b, uf_b, w1a, w1b, b1r)

    y2, st2 = pl.pallas_call(
        _k2,
        grid=(_B, _NT),
        in_specs=[
            pl.BlockSpec((1, _TN, _C2), lambda b, n: (b, n, 0)),
            const2((2, _C2)),
            const2((1, _C2)),
            const2((1, _C2)),
            const2((_C2, _C2)),
            const2((_C2, 1)),
        ],
        out_specs=[
            pl.BlockSpec((1, _C2, _TN), lambda b, n: (b, 0, n)),
            const2((_C2, 2)),
        ],
        out_shape=[
            jax.ShapeDtypeStruct((_B, _C2, _N), jnp.bfloat16),
            jax.ShapeDtypeStruct((_C2, 2), jnp.float32),
        ],
    )(y1t, st1, g1r, be1r, w2, b2r)

    out = pl.pallas_call(
        _k3,
        grid=(_B, _NT),
        in_specs=[
            pl.BlockSpec((1, _C2, _TN), lambda b, n: (b, 0, n)),
            const2((_C2, 2)),
            const2((_C2, 1)),
            const2((_C2, 1)),
        ],
        out_specs=pl.BlockSpec((1, _C2, _TN), lambda b, n: (b, 0, n)),
        out_shape=jax.ShapeDtypeStruct((_B, _C2, _N), jnp.float32),
    )(y2, st2, g2r, be2r)

    return out


# speedup vs baseline: 18.6699x; 1.0525x over previous
import jax
import jax.numpy as jnp
from jax.experimental import pallas as pl
from jax.experimental.pallas import tpu as pltpu

_B, _N, _M = 16, 4096, 1024
_C1, _C2 = 256, 512
_TN = 512
_NT = _N // _TN
_EPS_BN = 1e-5
_CNT = float(_B * _N)


def _k1(u_ref, kt_ref, kf_ref, uf_ref, w1a_ref, w1b_ref, b1_ref,
        y1_ref, st_ref):
    b = pl.program_id(0)
    nt = pl.program_id(1)

    u = u_ref[0]
    kt = kt_ref[0]
    d2 = jnp.zeros((_TN, _M), jnp.float32)
    for c in range(3):
        diff = u[:, c:c + 1] - kt[c:c + 1, :]
        d2 = d2 + diff * diff

    iota = jax.lax.broadcasted_iota(jnp.int32, (_TN, _M), 1)
    bits = jax.lax.bitcast_convert_type(d2, jnp.int32)
    v = (bits & jnp.int32(~1023)) | iota
    cur = v
    ms = []
    for k in range(3):
        m = jnp.min(cur, axis=1, keepdims=True)
        ms.append(m)
        if k < 2:
            cur = jnp.where(cur == m, jnp.int32(0x7FFFFFFF), cur)
    recips = []
    for k in range(3):
        d2k = jax.lax.bitcast_convert_type(ms[k] & jnp.int32(~1023),
                                           jnp.float32)
        recips.append(1.0 / (jnp.sqrt(d2k) + 1e-8))
    rsum = recips[0] + recips[1] + recips[2]
    zero = jnp.zeros((), jnp.float32)
    s = jnp.where(v == ms[0], recips[0] / rsum, zero)
    s = s + jnp.where(v == ms[1], recips[1] / rsum, zero)
    s = s + jnp.where(v == ms[2], recips[2] / rsum, zero)
    sb = s.astype(jnp.bfloat16)

    kf = kf_ref[0]
    interp = jax.lax.dot_general(sb, kf, (((1,), (1,)), ((), ())),
                                 preferred_element_type=jnp.float32)
    y1 = jax.lax.dot_general(interp.astype(jnp.bfloat16), w1a_ref[...],
                             (((1,), (1,)), ((), ())),
                             preferred_element_type=jnp.float32)
    uf = uf_ref[0]
    y1 = y1 + jax.lax.dot_general(uf, w1b_ref[...],
                                  (((0,), (1,)), ((), ())),
                                  preferred_element_type=jnp.float32)
    y1 = y1 + b1_ref[...]

    @pl.when(jnp.logical_and(b == 0, nt == 0))
    def _():
        st_ref[...] = jnp.zeros_like(st_ref)

    st_ref[0:1, :] += jnp.sum(y1, axis=0, keepdims=True)
    st_ref[1:2, :] += jnp.sum(y1 * y1, axis=0, keepdims=True)
    y1_ref[0] = y1.astype(jnp.bfloat16)


def _k2(y1_ref, st1_ref, g1_ref, be1_ref, w2_ref, b2_ref,
        y2_ref, st_ref):
    b = pl.program_id(0)
    nt = pl.program_id(1)

    mean = st1_ref[0:1, :] * (1.0 / _CNT)
    var = st1_ref[1:2, :] * (1.0 / _CNT) - mean * mean
    a1 = g1_ref[...] * jax.lax.rsqrt(var + _EPS_BN)
    c1 = be1_ref[...] - mean * a1

    y1 = y1_ref[0].astype(jnp.float32)
    h1 = jnp.maximum(a1 * y1 + c1, 0.0).astype(jnp.bfloat16)
    y2 = jax.lax.dot_general(w2_ref[...], h1, (((1,), (1,)), ((), ())),
                             preferred_element_type=jnp.float32)
    y2 = y2 + b2_ref[...]

    @pl.when(jnp.logical_and(b == 0, nt == 0))
    def _():
        st_ref[...] = jnp.zeros_like(st_ref)

    st_ref[:, 0:1] += jnp.sum(y2, axis=1, keepdims=True)
    st_ref[:, 1:2] += jnp.sum(y2 * y2, axis=1, keepdims=True)
    y2_ref[0] = y2.astype(jnp.bfloat16)


def _k3(y2_ref, st2_ref, g2_ref, be2_ref, out_ref):
    mean = st2_ref[:, 0:1] * (1.0 / _CNT)
    var = st2_ref[:, 1:2] * (1.0 / _CNT) - mean * mean
    a2 = g2_ref[...] * jax.lax.rsqrt(var + _EPS_BN)
    c2 = be2_ref[...] - mean * a2
    y2 = y2_ref[0].astype(jnp.float32)
    out_ref[0] = jnp.maximum(a2 * y2 + c2, 0.0)


def kernel(unknown, known, unknow_feats, known_feats,
           W1, b1, gamma1, beta1, W2, b2, gamma2, beta2):
    known_t = jnp.swapaxes(known, 1, 2)
    kf_b = known_feats.astype(jnp.bfloat16)
    uf_b = unknow_feats.astype(jnp.bfloat16)
    w1a = W1[:, :_C2].astype(jnp.bfloat16)
    w1b = W1[:, _C2:].astype(jnp.bfloat16)
    w2 = W2.astype(jnp.bfloat16)
    b1r = b1.reshape(1, _C2)
    g1r = gamma1.reshape(1, _C2)
    be1r = beta1.reshape(1, _C2)
    b2r = b2.reshape(_C2, 1)
    g2r = gamma2.reshape(_C2, 1)
    be2r = beta2.reshape(_C2, 1)

    const2 = lambda bs: pl.BlockSpec(bs, lambda b, n: (0, 0))

    y1t, st1 = pl.pallas_call(
        _k1,
        grid=(_B, _NT),
        in_specs=[
            pl.BlockSpec((1, _TN, 3), lambda b, n: (b, n, 0)),
            pl.BlockSpec((1, 3, _M), lambda b, n: (b, 0, 0)),
            pl.BlockSpec((1, _C2, _M), lambda b, n: (b, 0, 0)),
            pl.BlockSpec((1, _C1, _TN), lambda b, n: (b, 0, n)),
            const2((_C2, _C2)),
            const2((_C2, _C1)),
            const2((1, _C2)),
        ],
        out_specs=[
            pl.BlockSpec((1, _TN, _C2), lambda b, n: (b, n, 0)),
            const2((2, _C2)),
        ],
        out_shape=[
            jax.ShapeDtypeStruct((_B, _N, _C2), jnp.bfloat16),
            jax.ShapeDtypeStruct((2, _C2), jnp.float32),
        ],
    )(unknown, known_t, kf_b, uf_b, w1a, w1b, b1r)

    y2, st2 = pl.pallas_call(
        _k2,
        grid=(_B, _NT),
        in_specs=[
            pl.BlockSpec((1, _TN, _C2), lambda b, n: (b, n, 0)),
            const2((2, _C2)),
            const2((1, _C2)),
            const2((1, _C2)),
            const2((_C2, _C2)),
            const2((_C2, 1)),
        ],
        out_specs=[
            pl.BlockSpec((1, _C2, _TN), lambda b, n: (b, 0, n)),
            const2((_C2, 2)),
        ],
        out_shape=[
            jax.ShapeDtypeStruct((_B, _C2, _N), jnp.bfloat16),
            jax.ShapeDtypeStruct((_C2, 2), jnp.float32),
        ],
    )(y1t, st1, g1r, be1r, w2, b2r)

    out = pl.pallas_call(
        _k3,
        grid=(_B, _NT),
        in_specs=[
            pl.BlockSpec((1, _C2, _TN), lambda b, n: (b, 0, n)),
            const2((_C2, 2)),
            const2((_C2, 1)),
            const2((_C2, 1)),
        ],
        out_specs=pl.BlockSpec((1, _C2, _TN), lambda b, n: (b, 0, n)),
        out_shape=jax.ShapeDtypeStruct((_B, _C2, _N), jnp.float32),
    )(y2, st2, g2r, be2r)

    return out


# speedup vs baseline: 20.8007x; 1.1141x over previous
import jax
import jax.numpy as jnp
from jax.experimental import pallas as pl
from jax.experimental.pallas import tpu as pltpu

_B, _N, _M = 16, 4096, 1024
_C1, _C2 = 256, 512
_TN = 512
_NT = _N // _TN
_EPS_BN = 1e-5
_CNT = float(_B * _N)


def _k1(u_ref, kt_ref, kf_ref, uf_ref, w1a_ref, w1b_ref, b1_ref,
        y1_ref, st_ref):
    b = pl.program_id(0)
    nt = pl.program_id(1)

    u = u_ref[0]
    kt = kt_ref[0]
    d2 = jnp.zeros((_TN, _M), jnp.float32)
    for c in range(3):
        diff = u[:, c:c + 1] - kt[c:c + 1, :]
        d2 = d2 + diff * diff

    cur = d2
    ms = []
    for k in range(3):
        m = jnp.min(cur, axis=1, keepdims=True)
        ms.append(m)
        if k < 2:
            cur = jnp.where(cur == m, jnp.float32(jnp.inf), cur)
    recips = []
    for k in range(3):
        d2k = jnp.maximum(ms[k], 0.0)
        recips.append(1.0 / (jnp.sqrt(d2k) + 1e-8))
    rsum = recips[0] + recips[1] + recips[2]
    zero = jnp.zeros((), jnp.float32)
    s = jnp.where(d2 == ms[0], recips[0] / rsum, zero)
    s = s + jnp.where(d2 == ms[1], recips[1] / rsum, zero)
    s = s + jnp.where(d2 == ms[2], recips[2] / rsum, zero)
    sb = s.astype(jnp.bfloat16)

    kf = kf_ref[0]
    interp = jax.lax.dot_general(sb, kf, (((1,), (1,)), ((), ())),
                                 preferred_element_type=jnp.float32)
    y1 = jax.lax.dot_general(interp.astype(jnp.bfloat16), w1a_ref[...],
                             (((1,), (1,)), ((), ())),
                             preferred_element_type=jnp.float32)
    uf = uf_ref[0]
    y1 = y1 + jax.lax.dot_general(uf, w1b_ref[...],
                                  (((0,), (1,)), ((), ())),
                                  preferred_element_type=jnp.float32)
    y1 = y1 + b1_ref[...]

    @pl.when(jnp.logical_and(b == 0, nt == 0))
    def _():
        st_ref[...] = jnp.zeros_like(st_ref)

    st_ref[0:1, :] += jnp.sum(y1, axis=0, keepdims=True)
    st_ref[1:2, :] += jnp.sum(y1 * y1, axis=0, keepdims=True)
    y1_ref[0] = y1.astype(jnp.bfloat16)


def _k2(y1_ref, st1_ref, g1_ref, be1_ref, w2_ref, b2_ref,
        y2_ref, st_ref):
    b = pl.program_id(0)
    nt = pl.program_id(1)

    mean = st1_ref[0:1, :] * (1.0 / _CNT)
    var = st1_ref[1:2, :] * (1.0 / _CNT) - mean * mean
    a1 = g1_ref[...] * jax.lax.rsqrt(var + _EPS_BN)
    c1 = be1_ref[...] - mean * a1

    y1 = y1_ref[0].astype(jnp.float32)
    h1 = jnp.maximum(a1 * y1 + c1, 0.0).astype(jnp.bfloat16)
    y2 = jax.lax.dot_general(w2_ref[...], h1, (((1,), (1,)), ((), ())),
                             preferred_element_type=jnp.float32)
    y2 = y2 + b2_ref[...]

    @pl.when(jnp.logical_and(b == 0, nt == 0))
    def _():
        st_ref[...] = jnp.zeros_like(st_ref)

    st_ref[:, 0:1] += jnp.sum(y2, axis=1, keepdims=True)
    st_ref[:, 1:2] += jnp.sum(y2 * y2, axis=1, keepdims=True)
    y2_ref[0] = y2.astype(jnp.bfloat16)


def _k3(y2_ref, st2_ref, g2_ref, be2_ref, out_ref):
    mean = st2_ref[:, 0:1] * (1.0 / _CNT)
    var = st2_ref[:, 1:2] * (1.0 / _CNT) - mean * mean
    a2 = g2_ref[...] * jax.lax.rsqrt(var + _EPS_BN)
    c2 = be2_ref[...] - mean * a2
    y2 = y2_ref[0].astype(jnp.float32)
    out_ref[0] = jnp.maximum(a2 * y2 + c2, 0.0)


def kernel(unknown, known, unknow_feats, known_feats,
           W1, b1, gamma1, beta1, W2, b2, gamma2, beta2):
    known_t = jnp.swapaxes(known, 1, 2)
    kf_b = known_feats.astype(jnp.bfloat16)
    uf_b = unknow_feats.astype(jnp.bfloat16)
    w1a = W1[:, :_C2].astype(jnp.bfloat16)
    w1b = W1[:, _C2:].astype(jnp.bfloat16)
    w2 = W2.astype(jnp.bfloat16)
    b1r = b1.reshape(1, _C2)
    g1r = gamma1.reshape(1, _C2)
    be1r = beta1.reshape(1, _C2)
    b2r = b2.reshape(_C2, 1)
    g2r = gamma2.reshape(_C2, 1)
    be2r = beta2.reshape(_C2, 1)

    const2 = lambda bs: pl.BlockSpec(bs, lambda b, n: (0, 0))

    y1t, st1 = pl.pallas_call(
        _k1,
        grid=(_B, _NT),
        in_specs=[
            pl.BlockSpec((1, _TN, 3), lambda b, n: (b, n, 0)),
            pl.BlockSpec((1, 3, _M), lambda b, n: (b, 0, 0)),
            pl.BlockSpec((1, _C2, _M), lambda b, n: (b, 0, 0)),
            pl.BlockSpec((1, _C1, _TN), lambda b, n: (b, 0, n)),
            const2((_C2, _C2)),
            const2((_C2, _C1)),
            const2((1, _C2)),
        ],
        out_specs=[
            pl.BlockSpec((1, _TN, _C2), lambda b, n: (b, n, 0)),
            const2((2, _C2)),
        ],
        out_shape=[
            jax.ShapeDtypeStruct((_B, _N, _C2), jnp.bfloat16),
            jax.ShapeDtypeStruct((2, _C2), jnp.float32),
        ],
    )(unknown, known_t, kf_b, uf_b, w1a, w1b, b1r)

    y2, st2 = pl.pallas_call(
        _k2,
        grid=(_B, _NT),
        in_specs=[
            pl.BlockSpec((1, _TN, _C2), lambda b, n: (b, n, 0)),
            const2((2, _C2)),
            const2((1, _C2)),
            const2((1, _C2)),
            const2((_C2, _C2)),
            const2((_C2, 1)),
        ],
        out_specs=[
            pl.BlockSpec((1, _C2, _TN), lambda b, n: (b, 0, n)),
            const2((_C2, 2)),
        ],
        out_shape=[
            jax.ShapeDtypeStruct((_B, _C2, _N), jnp.bfloat16),
            jax.ShapeDtypeStruct((_C2, 2), jnp.float32),
        ],
    )(y1t, st1, g1r, be1r, w2, b2r)

    out = pl.pallas_call(
        _k3,
        grid=(_B, _NT),
        in_specs=[
            pl.BlockSpec((1, _C2, _TN), lambda b, n: (b, 0, n)),
            const2((_C2, 2)),
            const2((_C2, 1)),
            const2((_C2, 1)),
        ],
        out_specs=pl.BlockSpec((1, _C2, _TN), lambda b, n: (b, 0, n)),
        out_shape=jax.ShapeDtypeStruct((_B, _C2, _N), jnp.float32),
    )(y2, st2, g2r, be2r)

    return out
